# baseline (speedup 1.0000x reference)
import functools

import jax
import jax.numpy as jnp
from jax import lax
from jax.experimental import pallas as pl
from jax.experimental.pallas import tpu as pltpu
from jax.experimental.pallas import tpu_sc as plsc

D = 32
H = 50
NC = 2
NS = 16
NW = NC * NS
CH = 2
IPC = CH * H
NBUF = 8


def _build(batch):
    rows_w = batch // NW
    steps = rows_w // CH
    main = steps - NBUF
    assert main % NBUF == 0
    mesh = plsc.VectorSubcoreMesh(core_axis_name="c", subcore_axis_name="s")

    @functools.partial(
        pl.kernel,
        mesh=mesh,
        out_type=jax.ShapeDtypeStruct((batch, D), jnp.float32),
        compiler_params=pltpu.CompilerParams(use_tc_tiling_on_sc=False),
        scratch_types=[
            pltpu.VMEM((steps, IPC), jnp.int32),
            pltpu.VMEM((NBUF, IPC, D), jnp.float32),
            pltpu.VMEM((rows_w, D), jnp.float32),
        ] + [pltpu.SemaphoreType.DMA] * NBUF,
    )
    def k(idx_hbm, w_hbm, out_hbm, idx_v, buf, out_v, *sems):
        wid = lax.axis_index("s") * NC + lax.axis_index("c")
        pltpu.sync_copy(idx_hbm.at[pl.ds(wid * steps, steps)], idx_v)

        def start(chunk, b):
            pltpu.async_copy(w_hbm.at[idx_v.at[chunk]], buf.at[b], sems[b])

        def wait(b):
            pltpu.make_async_copy(w_hbm.at[idx_v.at[0]], buf.at[b], sems[b]).wait()

        def compute(chunk, b):
            for r in range(CH):
                acc0 = buf[b, r * H, pl.ds(0, 16)]
                acc1 = buf[b, r * H, pl.ds(16, 16)]
                for j in range(1, H):
                    acc0 = acc0 + buf[b, r * H + j, pl.ds(0, 16)]
                    acc1 = acc1 + buf[b, r * H + j, pl.ds(16, 16)]
                orow = out_v.at[chunk * CH + r]
                orow[pl.ds(0, 16)] = acc0
                orow[pl.ds(16, 16)] = acc1

        for b in range(NBUF):
            start(b, b)

        def body(i, carry):
            g = i * NBUF
            for b in range(NBUF):
                chunk = g + b
                wait(b)
                compute(chunk, b)
                start(chunk + NBUF, b)
            return carry

        lax.fori_loop(0, main // NBUF, body, 0)

        for b in range(NBUF):
            wait(b)
            compute(main + b, b)

        pltpu.sync_copy(out_v, out_hbm.at[pl.ds(wid * rows_w, rows_w)])

    return k


def kernel(inputs, W):
    batch, hist = inputs.shape
    assert hist == H and batch % (NW * CH) == 0
    idx2d = inputs.astype(jnp.int32).reshape(batch // CH, IPC)
    return _build(batch)(idx2d, W)


# speedup vs baseline: 1.0395x; 1.0395x over previous
import functools

import jax
import jax.numpy as jnp
from jax import lax
from jax.experimental import pallas as pl
from jax.experimental.pallas import tpu as pltpu
from jax.experimental.pallas import tpu_sc as plsc

D = 32
H = 50
NC = 2
NS = 16
NW = NC * NS
CH = 2
IPC = CH * H
NBUF = 4


def _build(batch):
    rows_w = batch // NW
    steps = rows_w // CH
    main = steps - NBUF
    assert main % NBUF == 0
    mesh = plsc.VectorSubcoreMesh(core_axis_name="c", subcore_axis_name="s")

    @functools.partial(
        pl.kernel,
        mesh=mesh,
        out_type=jax.ShapeDtypeStruct((batch, D), jnp.float32),
        compiler_params=pltpu.CompilerParams(use_tc_tiling_on_sc=False),
        scratch_types=[
            pltpu.VMEM((steps, IPC), jnp.int32),
            pltpu.VMEM((NBUF, IPC, D), jnp.float32),
            pltpu.VMEM((rows_w, D), jnp.float32),
        ] + [pltpu.SemaphoreType.DMA] * NBUF,
    )
    def k(idx_hbm, w_hbm, out_hbm, idx_v, buf, out_v, *sems):
        wid = lax.axis_index("s") * NC + lax.axis_index("c")
        pltpu.sync_copy(idx_hbm.at[pl.ds(wid * steps, steps)], idx_v)

        def start(chunk, b):
            pltpu.async_copy(w_hbm.at[idx_v.at[chunk]], buf.at[b], sems[b])

        def wait(b):
            pltpu.make_async_copy(w_hbm.at[idx_v.at[0]], buf.at[b], sems[b]).wait()

        def compute(chunk, b):
            for r in range(CH):
                acc0 = buf[b, r * H, pl.ds(0, 16)]
                acc1 = buf[b, r * H, pl.ds(16, 16)]
                for j in range(1, H):
                    acc0 = acc0 + buf[b, r * H + j, pl.ds(0, 16)]
                    acc1 = acc1 + buf[b, r * H + j, pl.ds(16, 16)]
                orow = out_v.at[chunk * CH + r]
                orow[pl.ds(0, 16)] = acc0
                orow[pl.ds(16, 16)] = acc1

        for b in range(NBUF):
            start(b, b)

        def body(i, carry):
            g = i * NBUF
            for b in range(NBUF):
                chunk = g + b
                wait(b)
                compute(chunk, b)
                start(chunk + NBUF, b)
            return carry

        lax.fori_loop(0, main // NBUF, body, 0)

        for b in range(NBUF):
            wait(b)
            compute(main + b, b)

        pltpu.sync_copy(out_v, out_hbm.at[pl.ds(wid * rows_w, rows_w)])

    return k


def kernel(inputs, W):
    batch, hist = inputs.shape
    assert hist == H and batch % (NW * CH) == 0
    idx2d = inputs.astype(jnp.int32).reshape(batch // CH, IPC)
    return _build(batch)(idx2d, W)
